# baseline (speedup 1.0000x reference)
import functools

import jax
import jax.numpy as jnp
from jax.experimental import pallas as pl
from jax.experimental.pallas import tpu as pltpu

N = 50000
E = 800000
B = 8
LAT = 32
VLAT = 16

EBLK = 4000
NBLK = 2000


def _ln(x, g, b):
    m = jnp.mean(x, axis=-1, keepdims=True)
    v = jnp.mean((x - m) * (x - m), axis=-1, keepdims=True)
    return (x - m) / jnp.sqrt(v + 1e-5) * g + b


def _dot(a, b):
    return jnp.dot(a, b, preferred_element_type=jnp.float32)


def _enc_body(x_ref, w1, b1, w2, b2, g, bln, o_ref):
    h = jnp.maximum(_dot(x_ref[...], w1[...]) + b1[...], 0.0)
    o_ref[...] = _ln(_dot(h, w2[...]) + b2[...], g[...], bln[...])


def _full(a):
    return pl.BlockSpec(a.shape, lambda i: (0,) * a.ndim)


def _edge_encoder(edge_attr, p):
    w1, b1, w2, b2, g, bln = (p["W1"], p["b1"].reshape(1, -1), p["W2"],
                              p["b2"].reshape(1, -1), p["g"].reshape(1, -1),
                              p["bln"].reshape(1, -1))
    return pl.pallas_call(
        _enc_body,
        grid=(E // EBLK,),
        in_specs=[pl.BlockSpec((EBLK, 8), lambda i: (i, 0))]
        + [_full(a) for a in (w1, b1, w2, b2, g, bln)],
        out_specs=pl.BlockSpec((EBLK, LAT), lambda i: (i, 0)),
        out_shape=jax.ShapeDtypeStruct((E, LAT), jnp.float32),
    )(edge_attr, w1, b1, w2, b2, g, bln)


def _node_enc_body(x_ref, w1, b1, w2, b2, g, bln, ws, wd, h_ref, hps_ref, hpd_ref):
    h = jnp.maximum(_dot(x_ref[...], w1[...]) + b1[...], 0.0)
    h = _ln(_dot(h, w2[...]) + b2[...], g[...], bln[...])
    h_ref[...] = h
    hps_ref[...] = _dot(h, ws[...])
    hpd_ref[...] = _dot(h, wd[...])


def _node_encoder(y, p, w1s, w1d):
    w1, b1, w2, b2, g, bln = (p["W1"], p["b1"].reshape(1, -1), p["W2"],
                              p["b2"].reshape(1, -1), p["g"].reshape(1, -1),
                              p["bln"].reshape(1, -1))
    blk = lambda c: pl.BlockSpec((NBLK, c), lambda i: (i, 0))
    return pl.pallas_call(
        _node_enc_body,
        grid=(N // NBLK,),
        in_specs=[blk(3)] + [_full(a) for a in (w1, b1, w2, b2, g, bln, w1s, w1d)],
        out_specs=[blk(LAT)] * 3,
        out_shape=[jax.ShapeDtypeStruct((N, LAT), jnp.float32)] * 3,
    )(y, w1, b1, w2, b2, g, bln, w1s, w1d)


def _edge_mlp_body(e_ref, g_ref, w1e, b1, w2, b2, gg, bln, o_ref):
    hid = jnp.maximum(_dot(e_ref[...], w1e[...]) + g_ref[...] + b1[...], 0.0)
    o_ref[...] = e_ref[...] + _ln(_dot(hid, w2[...]) + b2[...], gg[...], bln[...])


def _edge_mlp(e, gsum, p, w1e):
    b1, w2, b2, g, bln = (p["b1"].reshape(1, -1), p["W2"], p["b2"].reshape(1, -1),
                          p["g"].reshape(1, -1), p["bln"].reshape(1, -1))
    blk = pl.BlockSpec((EBLK, LAT), lambda i: (i, 0))
    return pl.pallas_call(
        _edge_mlp_body,
        grid=(E // EBLK,),
        in_specs=[blk, blk] + [_full(a) for a in (w1e, b1, w2, b2, g, bln)],
        out_specs=blk,
        out_shape=jax.ShapeDtypeStruct((E, LAT), jnp.float32),
    )(e, gsum, w1e, b1, w2, b2, g, bln)


def _node_mlp_body(h_ref, agg_ref, w1h, w1a, b1, w2, b2, g, bln, ws, wd,
                   h_ref_o, hps_ref, hpd_ref):
    hid = jnp.maximum(_dot(h_ref[...], w1h[...]) + _dot(agg_ref[...], w1a[...])
                      + b1[...], 0.0)
    hn = h_ref[...] + _ln(_dot(hid, w2[...]) + b2[...], g[...], bln[...])
    h_ref_o[...] = hn
    hps_ref[...] = _dot(hn, ws[...])
    hpd_ref[...] = _dot(hn, wd[...])


def _node_mlp_last_body(h_ref, agg_ref, w1h, w1a, b1, w2, b2, g, bln, h_ref_o):
    hid = jnp.maximum(_dot(h_ref[...], w1h[...]) + _dot(agg_ref[...], w1a[...])
                      + b1[...], 0.0)
    h_ref_o[...] = h_ref[...] + _ln(_dot(hid, w2[...]) + b2[...], g[...], bln[...])


def _node_mlp(h, agg, p, w1s=None, w1d=None):
    w1 = p["W1"]
    w1h, w1a = w1[:LAT], w1[LAT:]
    b1, w2, b2, g, bln = (p["b1"].reshape(1, -1), p["W2"], p["b2"].reshape(1, -1),
                          p["g"].reshape(1, -1), p["bln"].reshape(1, -1))
    blk = pl.BlockSpec((NBLK, LAT), lambda i: (i, 0))
    if w1s is None:
        return pl.pallas_call(
            _node_mlp_last_body,
            grid=(N // NBLK,),
            in_specs=[blk, blk] + [_full(a) for a in (w1h, w1a, b1, w2, b2, g, bln)],
            out_specs=blk,
            out_shape=jax.ShapeDtypeStruct((N, LAT), jnp.float32),
        )(h, agg, w1h, w1a, b1, w2, b2, g, bln)
    return pl.pallas_call(
        _node_mlp_body,
        grid=(N // NBLK,),
        in_specs=[blk, blk] + [_full(a) for a in (w1h, w1a, b1, w2, b2, g, bln, w1s, w1d)],
        out_specs=[blk] * 3,
        out_shape=[jax.ShapeDtypeStruct((N, LAT), jnp.float32)] * 3,
    )(h, agg, w1h, w1a, b1, w2, b2, g, bln, w1s, w1d)


def _pool_max_body(h_ref, oh_ref, gwb, gmax_ref, m_acc):
    i = pl.program_id(0)

    @pl.when(i == 0)
    def _init():
        m_acc[...] = jnp.full((1, B), -1e30, jnp.float32)

    gate = _dot(h_ref[...], gwb[...])
    masked = jnp.where(oh_ref[...] > 0.5, gate, -1e30)
    m_acc[...] = jnp.maximum(m_acc[...], jnp.max(masked, axis=0, keepdims=True))

    @pl.when(i == pl.num_programs(0) - 1)
    def _out():
        gmax_ref[...] = m_acc[...]


def _pool_max(h, onehot, gwb):
    blk = pl.BlockSpec((NBLK, LAT), lambda i: (i, 0))
    bblk = pl.BlockSpec((NBLK, B), lambda i: (i, 0))
    return pl.pallas_call(
        _pool_max_body,
        grid=(N // NBLK,),
        in_specs=[blk, bblk, _full(gwb)],
        out_specs=_full(jnp.zeros((1, B))),
        out_shape=jax.ShapeDtypeStruct((1, B), jnp.float32),
        scratch_shapes=[pltpu.VMEM((1, B), jnp.float32)],
    )(h, onehot, gwb)


def _pool_final_body(h_ref, oh_ref, gmax_ref, gwb, mu_w, mu_b, lv_w, lv_b, eps,
                     z_ref, mu_ref, lv_ref, den_acc, wacc):
    i = pl.program_id(0)

    @pl.when(i == 0)
    def _init():
        den_acc[...] = jnp.zeros((B, LAT), jnp.float32)
        wacc[...] = jnp.zeros((B, LAT), jnp.float32)

    h = h_ref[...]
    gate = _dot(h, gwb[...])
    ex = jnp.where(oh_ref[...] > 0.5,
                   jnp.exp(gate - gmax_ref[...]), 0.0)
    ones = jnp.ones((NBLK, LAT), jnp.float32)
    den_acc[...] += jax.lax.dot_general(ex, ones, (((0,), (0,)), ((), ())),
                                        preferred_element_type=jnp.float32)
    wacc[...] += jax.lax.dot_general(ex, h, (((0,), (0,)), ((), ())),
                                     preferred_element_type=jnp.float32)

    @pl.when(i == pl.num_programs(0) - 1)
    def _out():
        hg = wacc[...] / (den_acc[...] + 1e-16)
        mu = _dot(hg, mu_w[...]) + mu_b[...]
        lv = _dot(hg, lv_w[...]) + lv_b[...]
        mu_ref[...] = mu
        lv_ref[...] = lv
        z_ref[...] = mu + jnp.exp(0.5 * lv) * eps[...]


def _pool_final(h, onehot, gmax, gwb, mu_w, mu_b, lv_w, lv_b, eps):
    blk = pl.BlockSpec((NBLK, LAT), lambda i: (i, 0))
    bblk = pl.BlockSpec((NBLK, B), lambda i: (i, 0))
    outs = [jax.ShapeDtypeStruct((B, VLAT), jnp.float32)] * 3
    return pl.pallas_call(
        _pool_final_body,
        grid=(N // NBLK,),
        in_specs=[blk, bblk] + [_full(a) for a in (gmax, gwb, mu_w, mu_b, lv_w, lv_b, eps)],
        out_specs=[_full(jnp.zeros((B, VLAT)))] * 3,
        out_shape=outs,
        scratch_shapes=[pltpu.VMEM((B, LAT), jnp.float32)] * 2,
    )(h, onehot, gmax, gwb, mu_w, mu_b, lv_w, lv_b, eps)


def kernel(y, edge_index, edge_attr, batch, params):
    p = params
    src = edge_index[0]
    dst = edge_index[1]

    w1 = [p["mp"][l]["edge_mlp"]["W1"] for l in range(2)]
    w1e = [w[:LAT] for w in w1]
    w1s = [w[LAT:2 * LAT] for w in w1]
    w1d = [w[2 * LAT:] for w in w1]

    e = _edge_encoder(edge_attr, p["edge_enc"])
    h, hps, hpd = _node_encoder(y, p["node_enc"], w1s[0], w1d[0])

    for l in range(2):
        gsum = jnp.take(hps, src, axis=0) + jnp.take(hpd, dst, axis=0)
        e = _edge_mlp(e, gsum, p["mp"][l]["edge_mlp"], w1e[l])
        agg = jax.ops.segment_sum(e, dst, num_segments=N)
        if l == 0:
            h, hps, hpd = _node_mlp(h, agg, p["mp"][l]["node_mlp"],
                                    w1s[1], w1d[1])
        else:
            h = _node_mlp(h, agg, p["mp"][l]["node_mlp"])

    onehot = (batch[:, None] == jnp.arange(B, dtype=batch.dtype)[None, :]
              ).astype(jnp.float32)
    gwb = jnp.tile(p["gate_W"], (1, B))
    gmax = _pool_max(h, onehot, gwb)
    eps = jax.random.normal(jax.random.key(42), (B, VLAT), dtype=jnp.float32)
    z, mu, lv = _pool_final(h, onehot, gmax, gwb,
                            p["mu_W"], p["mu_b"].reshape(1, -1),
                            p["lv_W"], p["lv_b"].reshape(1, -1), eps)
    return z, mu, lv


# speedup vs baseline: 1.9304x; 1.9304x over previous
import functools

import jax
import jax.numpy as jnp
from jax import lax
from jax.experimental import pallas as pl
from jax.experimental.pallas import tpu as pltpu
from jax.experimental.pallas import tpu_sc as plsc

N = 50000
E = 800000
B = 8
LAT = 32
VLAT = 16

EBLK = 4000
NBLK = 2000


def _ln(x, g, b):
    m = jnp.mean(x, axis=-1, keepdims=True)
    v = jnp.mean((x - m) * (x - m), axis=-1, keepdims=True)
    return (x - m) / jnp.sqrt(v + 1e-5) * g + b


def _dot(a, b):
    return jnp.dot(a, b, preferred_element_type=jnp.float32)


def _enc_body(x_ref, w1, b1, w2, b2, g, bln, o_ref):
    h = jnp.maximum(_dot(x_ref[...], w1[...]) + b1[...], 0.0)
    o_ref[...] = _ln(_dot(h, w2[...]) + b2[...], g[...], bln[...])


def _full(a):
    return pl.BlockSpec(a.shape, lambda i: (0,) * a.ndim)


def _edge_encoder(edge_attr, p):
    w1, b1, w2, b2, g, bln = (p["W1"], p["b1"].reshape(1, -1), p["W2"],
                              p["b2"].reshape(1, -1), p["g"].reshape(1, -1),
                              p["bln"].reshape(1, -1))
    return pl.pallas_call(
        _enc_body,
        grid=(E // EBLK,),
        in_specs=[pl.BlockSpec((EBLK, 8), lambda i: (i, 0))]
        + [_full(a) for a in (w1, b1, w2, b2, g, bln)],
        out_specs=pl.BlockSpec((EBLK, LAT), lambda i: (i, 0)),
        out_shape=jax.ShapeDtypeStruct((E, LAT), jnp.float32),
    )(edge_attr, w1, b1, w2, b2, g, bln)


def _node_enc_body(x_ref, w1, b1, w2, b2, g, bln, wcat, h_ref, t_ref):
    h = jnp.maximum(_dot(x_ref[...], w1[...]) + b1[...], 0.0)
    h = _ln(_dot(h, w2[...]) + b2[...], g[...], bln[...])
    h_ref[...] = h
    t_ref[...] = _dot(h, wcat[...])


def _node_encoder(y, p, wcat):
    w1, b1, w2, b2, g, bln = (p["W1"], p["b1"].reshape(1, -1), p["W2"],
                              p["b2"].reshape(1, -1), p["g"].reshape(1, -1),
                              p["bln"].reshape(1, -1))
    blk = lambda c: pl.BlockSpec((NBLK, c), lambda i: (i, 0))
    return pl.pallas_call(
        _node_enc_body,
        grid=(N // NBLK,),
        in_specs=[blk(3)] + [_full(a) for a in (w1, b1, w2, b2, g, bln, wcat)],
        out_specs=[blk(LAT), blk(4 * LAT)],
        out_shape=[jax.ShapeDtypeStruct((N, LAT), jnp.float32),
                   jax.ShapeDtypeStruct((N, 4 * LAT), jnp.float32)],
    )(y, w1, b1, w2, b2, g, bln, wcat)


def _edge_mlp_body(e_ref, g_ref, w1e, b1, w2, b2, gg, bln, o_ref):
    hid = jnp.maximum(_dot(e_ref[...], w1e[...]) + g_ref[...] + b1[...], 0.0)
    o_ref[...] = e_ref[...] + _ln(_dot(hid, w2[...]) + b2[...], gg[...], bln[...])


def _edge_mlp(e, gsum, p, w1e):
    b1, w2, b2, g, bln = (p["b1"].reshape(1, -1), p["W2"], p["b2"].reshape(1, -1),
                          p["g"].reshape(1, -1), p["bln"].reshape(1, -1))
    blk = pl.BlockSpec((EBLK, LAT), lambda i: (i, 0))
    return pl.pallas_call(
        _edge_mlp_body,
        grid=(E // EBLK,),
        in_specs=[blk, blk] + [_full(a) for a in (w1e, b1, w2, b2, g, bln)],
        out_specs=blk,
        out_shape=jax.ShapeDtypeStruct((E, LAT), jnp.float32),
    )(e, gsum, w1e, b1, w2, b2, g, bln)


def _node_mlp_body(h_ref, agg_ref, w1h, w1a, b1, w2, b2, g, bln, wcat,
                   h_ref_o, t_ref):
    agg = agg_ref[...]
    hid = jnp.maximum(_dot(h_ref[...], w1h[...]) + _dot(agg, w1a[...])
                      + b1[...], 0.0)
    hn = h_ref[...] + _ln(_dot(hid, w2[...]) + b2[...], g[...], bln[...])
    h_ref_o[...] = hn
    t_ref[...] = _dot(hn, wcat[...])


def _node_mlp_last_body(h_ref, agg_ref, w1h, w1a, b1, w2, b2, g, bln,
                        h_ref_o):
    agg = agg_ref[...]
    hid = jnp.maximum(_dot(h_ref[...], w1h[...]) + _dot(agg, w1a[...])
                      + b1[...], 0.0)
    h_ref_o[...] = h_ref[...] + _ln(_dot(hid, w2[...]) + b2[...], g[...], bln[...])


def _node_mlp(h, agg, p, wcat=None):
    w1 = p["W1"]
    w1h, w1a = w1[:LAT], w1[LAT:]
    b1, w2, b2, g, bln = (p["b1"].reshape(1, -1), p["W2"], p["b2"].reshape(1, -1),
                          p["g"].reshape(1, -1), p["bln"].reshape(1, -1))
    blk = pl.BlockSpec((NBLK, LAT), lambda i: (i, 0))
    if wcat is None:
        return pl.pallas_call(
            _node_mlp_last_body,
            grid=(N // NBLK,),
            in_specs=[blk, blk] + [_full(a) for a in (w1h, w1a, b1, w2, b2, g, bln)],
            out_specs=blk,
            out_shape=jax.ShapeDtypeStruct((N, LAT), jnp.float32),
        )(h, agg, w1h, w1a, b1, w2, b2, g, bln)
    return pl.pallas_call(
        _node_mlp_body,
        grid=(N // NBLK,),
        in_specs=[blk, blk] + [_full(a) for a in (w1h, w1a, b1, w2, b2, g, bln, wcat)],
        out_specs=[blk, pl.BlockSpec((NBLK, 4 * LAT), lambda i: (i, 0))],
        out_shape=[jax.ShapeDtypeStruct((N, LAT), jnp.float32),
                   jax.ShapeDtypeStruct((N, 4 * LAT), jnp.float32)],
    )(h, agg, w1h, w1a, b1, w2, b2, g, bln, wcat)


def _pool_max_body(h_ref, oh_ref, gwb, gmax_ref, m_acc):
    i = pl.program_id(0)

    @pl.when(i == 0)
    def _init():
        m_acc[...] = jnp.full((1, B), -1e30, jnp.float32)

    gate = _dot(h_ref[...], gwb[...])
    masked = jnp.where(oh_ref[...] > 0.5, gate, -1e30)
    m_acc[...] = jnp.maximum(m_acc[...], jnp.max(masked, axis=0, keepdims=True))

    @pl.when(i == pl.num_programs(0) - 1)
    def _out():
        gmax_ref[...] = m_acc[...]


def _pool_max(h, onehot, gwb):
    blk = pl.BlockSpec((NBLK, LAT), lambda i: (i, 0))
    bblk = pl.BlockSpec((NBLK, B), lambda i: (i, 0))
    return pl.pallas_call(
        _pool_max_body,
        grid=(N // NBLK,),
        in_specs=[blk, bblk, _full(gwb)],
        out_specs=_full(jnp.zeros((1, B))),
        out_shape=jax.ShapeDtypeStruct((1, B), jnp.float32),
        scratch_shapes=[pltpu.VMEM((1, B), jnp.float32)],
    )(h, onehot, gwb)


def _pool_final_body(h_ref, oh_ref, gmax_ref, gwb, mu_w, mu_b, lv_w, lv_b, eps,
                     z_ref, mu_ref, lv_ref, den_acc, wacc):
    i = pl.program_id(0)

    @pl.when(i == 0)
    def _init():
        den_acc[...] = jnp.zeros((B, LAT), jnp.float32)
        wacc[...] = jnp.zeros((B, LAT), jnp.float32)

    h = h_ref[...]
    gate = _dot(h, gwb[...])
    ex = jnp.where(oh_ref[...] > 0.5,
                   jnp.exp(gate - gmax_ref[...]), 0.0)
    ones = jnp.ones((NBLK, LAT), jnp.float32)
    den_acc[...] += jax.lax.dot_general(ex, ones, (((0,), (0,)), ((), ())),
                                        preferred_element_type=jnp.float32)
    wacc[...] += jax.lax.dot_general(ex, h, (((0,), (0,)), ((), ())),
                                     preferred_element_type=jnp.float32)

    @pl.when(i == pl.num_programs(0) - 1)
    def _out():
        hg = wacc[...] / (den_acc[...] + 1e-16)
        mu = _dot(hg, mu_w[...]) + mu_b[...]
        lv = _dot(hg, lv_w[...]) + lv_b[...]
        mu_ref[...] = mu
        lv_ref[...] = lv
        z_ref[...] = mu + jnp.exp(0.5 * lv) * eps[...]


def _pool_final(h, onehot, gmax, gwb, mu_w, mu_b, lv_w, lv_b, eps):
    blk = pl.BlockSpec((NBLK, LAT), lambda i: (i, 0))
    bblk = pl.BlockSpec((NBLK, B), lambda i: (i, 0))
    outs = [jax.ShapeDtypeStruct((B, VLAT), jnp.float32)] * 3
    return pl.pallas_call(
        _pool_final_body,
        grid=(N // NBLK,),
        in_specs=[blk, bblk] + [_full(a) for a in (gmax, gwb, mu_w, mu_b, lv_w, lv_b, eps)],
        out_specs=[_full(jnp.zeros((B, VLAT)))] * 3,
        out_shape=outs,
        scratch_shapes=[pltpu.VMEM((B, LAT), jnp.float32)] * 2,
    )(h, onehot, gmax, gwb, mu_w, mu_b, lv_w, lv_b, eps)


NC, NS = 2, 16
NW = NC * NS
EPW = E // NW
GC = 128
NFULL = EPW // GC
TAIL = EPW - NFULL * GC
KBUF = 3
NPW = N // NS

_MESH = plsc.VectorSubcoreMesh(core_axis_name="c", subcore_axis_name="s")


def _sc_gather(tab, src, dst):
    GB = 2

    @functools.partial(
        pl.kernel,
        out_type=jax.ShapeDtypeStruct((E, LAT), jnp.float32),
        mesh=_MESH,
        scratch_types=[pltpu.VMEM((GB * GC,), jnp.int32)] * 2
        + [pltpu.VMEM((GC, 4 * LAT), jnp.float32)] * (2 * GB)
        + [pltpu.VMEM((GC, LAT), jnp.float32)] * GB
        + [pltpu.SemaphoreType.DMA] * 2,
    )
    def k(tab_h, src_h, dst_h, gsum_h, *rest):
        isb, idb = rest[0], rest[1]
        abufs = rest[2:2 + 2 * GB]
        obufs = rest[2 + 2 * GB:2 + 3 * GB]
        sg, sw = rest[2 + 3 * GB], rest[3 + 3 * GB]
        wid = lax.axis_index("s") * NC + lax.axis_index("c")
        base = pl.multiple_of(wid * EPW, 8)

        def do_chunks(j0, sizes):
            n_idx = sum(sizes)
            off = pl.multiple_of(base + j0 * GC, 8)
            pltpu.sync_copy(src_h.at[pl.ds(off, n_idx)], isb.at[pl.ds(0, n_idx)])
            pltpu.sync_copy(dst_h.at[pl.ds(off, n_idx)], idb.at[pl.ds(0, n_idx)])
            gets = []
            for p, nrows in enumerate(sizes):
                gets.append(pltpu.async_copy(
                    tab_h.at[isb.at[pl.ds(p * GC, nrows)]],
                    abufs[2 * p].at[pl.ds(0, nrows)], sg))
                gets.append(pltpu.async_copy(
                    tab_h.at[idb.at[pl.ds(p * GC, nrows)]],
                    abufs[2 * p + 1].at[pl.ds(0, nrows)], sg))
            for d in gets:
                d.wait()
            for p, nrows in enumerate(sizes):
                sa, da = abufs[2 * p], abufs[2 * p + 1]
                ob = obufs[p]

                def addrow(r, _):
                    ob[r, pl.ds(0, 16)] = sa[r, pl.ds(0, 16)] + da[r, pl.ds(LAT, 16)]
                    ob[r, pl.ds(16, 16)] = sa[r, pl.ds(16, 16)] + da[r, pl.ds(LAT + 16, 16)]
                    return ()
                lax.fori_loop(0, nrows, addrow, (), unroll=8)
            puts = []
            for p, nrows in enumerate(sizes):
                poff = pl.multiple_of(base + (j0 + p) * GC, 8)
                puts.append(pltpu.async_copy(
                    obufs[p].at[pl.ds(0, nrows)],
                    gsum_h.at[pl.ds(poff, nrows)], sw))
            for d in puts:
                d.wait()

        def group(g, _):
            do_chunks(g * GB, (GC,) * GB)
            return ()

        lax.fori_loop(0, (NFULL - 1) // GB, group, ())
        do_chunks(NFULL - 1, (GC, TAIL))

    return k(tab, src, dst)


NCHUNK = E // GC
CPW = NCHUNK // NW
NEXTRA = NCHUNK - CPW * NW


NHALF = N // NC
APAD = 25088
DUMP = 25080
SROW = APAD // NS
WROW = 1568
WLAST = NHALF - 15 * WROW
CPT = NCHUNK // NS
XTRA = NCHUNK - CPT * NS
SK = 3


def _sc_scatter(e, dloc):
    @functools.partial(
        pl.kernel,
        out_type=jax.ShapeDtypeStruct((N, LAT), jnp.float32),
        mesh=_MESH,
        scratch_types=[pltpu.VMEM((GC, LAT), jnp.float32)],
    )
    def k(e_h, dl_h, out_h, *rest):
        vbufs = rest
        c = lax.axis_index("c")
        s = lax.axis_index("s")
        nbase = c * NHALF

        def zrow(r, _):
            z = jnp.zeros((16,), jnp.float32)
            vbufs[0][r, pl.ds(0, 16)] = z
            vbufs[0][r, pl.ds(16, 16)] = z
            return ()
        lax.fori_loop(0, GC, zrow, (), unroll=8)
        srow = pl.multiple_of(s * SROW, 8)
        if False:
            for q in range(SROW // GC):
                pltpu.sync_copy(vbufs[0], aggsh.at[pl.ds(srow + q * GC, GC)])
            pltpu.sync_copy(vbufs[0].at[pl.ds(0, SROW - (SROW // GC) * GC)],
                            aggsh.at[pl.ds(srow + (SROW // GC) * GC,
                                           SROW - (SROW // GC) * GC)])
            plsc.subcore_barrier()

        def do_chunks(j0, nch):
            gets = []
            for p in range(nch):
                gets.append(pltpu.async_copy(
                    dl_h.at[c].at[j0 + p], idxc[p], sg))
                eoff = pl.multiple_of((j0 + p) * GC, GC)
                gets.append(pltpu.async_copy(
                    e_h.at[pl.ds(eoff, GC)], vbufs[p], sg))
            for d in gets:
                d.wait()
            for p in range(nch):
                pltpu.sync_copy(vbufs[p], aggsh.at[idxc[p]], add=True)

        def group(g, _):
            do_chunks(s * CPT + g * SK, SK)
            return ()

        if True:
            pass
        else:
            lax.fori_loop(0, CPT // SK, group, ())

            @pl.when(s < XTRA)
            def _extra():
                do_chunks(NS * CPT + s, 1)

        def bounce(arow, orow, nrows):
            pltpu.sync_copy(vbufs[0].at[pl.ds(0, nrows)],
                            out_h.at[pl.ds(orow, nrows)])

        @pl.when(s < NS - 1)
        def _wout():
            for q in range(WROW // GC):
                bounce(srow + q * GC, nbase + s * WROW + q * GC, GC)
            rem = WROW - (WROW // GC) * GC
            bounce(srow + (WROW // GC) * GC,
                   nbase + s * WROW + (WROW // GC) * GC, rem)

        @pl.when(s == NS - 1)
        def _wlast():
            for q in range(WLAST // GC):
                bounce((NS - 1) * WROW + q * GC,
                       nbase + (NS - 1) * WROW + q * GC, GC)
            rem = WLAST - (WLAST // GC) * GC
            bounce((NS - 1) * WROW + (WLAST // GC) * GC,
                   nbase + (NS - 1) * WROW + (WLAST // GC) * GC, rem)

    return k(e, dloc)


def kernel(y, edge_index, edge_attr, batch, params):
    p = params
    src = edge_index[0]
    dst = edge_index[1]

    w1 = [p["mp"][l]["edge_mlp"]["W1"] for l in range(2)]
    w1e = [w[:LAT] for w in w1]
    wcat = [jnp.concatenate(
        [w[LAT:2 * LAT], w[2 * LAT:], jnp.zeros((LAT, 2 * LAT), jnp.float32)],
        axis=1) for w in w1]

    e = _edge_encoder(edge_attr, p["edge_enc"])
    h, tab = _node_encoder(y, p["node_enc"], wcat[0])

    loc = dst[None, :] - (jnp.arange(NC, dtype=jnp.int32) * NHALF)[:, None]
    dloc = jnp.where((loc >= 0) & (loc < NHALF), loc, DUMP
                     ).reshape(NC, NCHUNK, GC)
    for l in range(2):
        gsum = _sc_gather(tab, src, dst)
        e = _edge_mlp(e, gsum, p["mp"][l]["edge_mlp"], w1e[l])
        agg = jax.ops.segment_sum(e, dst, num_segments=N)
        if l == 0:
            h, tab = _node_mlp(h, agg, p["mp"][l]["node_mlp"], wcat[1])
        else:
            h = _node_mlp(h, agg, p["mp"][l]["node_mlp"])

    onehot = (batch[:, None] == jnp.arange(B, dtype=batch.dtype)[None, :]
              ).astype(jnp.float32)
    gwb = jnp.tile(p["gate_W"], (1, B))
    gmax = _pool_max(h, onehot, gwb)
    eps = jax.random.normal(jax.random.key(42), (B, VLAT), dtype=jnp.float32)
    z, mu, lv = _pool_final(h, onehot, gmax, gwb,
                            p["mu_W"], p["mu_b"].reshape(1, -1),
                            p["lv_W"], p["lv_b"].reshape(1, -1), eps)
    return z, mu, lv
